# baseline (speedup 1.0000x reference)
import functools

import jax
import jax.numpy as jnp
from jax import lax
from jax.experimental import pallas as pl
from jax.experimental.pallas import tpu as pltpu
from jax.experimental.pallas import tpu_sc as plsc

N = 10000
NPAD = 10112
ROWS_PER_TILE = NPAD // 16
E = 160000
BATCH = 88
NB = 120
EDGES_PER_TILE = NB * BATCH
EPAD = 16 * EDGES_PER_TILE
TRIPS = NB // 8
DIM = 512
G = 64
BR = 1000
NBLK = N // BR
BN_EPS = 1e-5


def _sc_segment_sum(C):
  mesh = plsc.VectorSubcoreMesh(core_axis_name="c", subcore_axis_name="s")
  cpc = C // 2

  @functools.partial(
      pl.kernel,
      out_type=jax.ShapeDtypeStruct((C * NPAD, 128), jnp.float32),
      mesh=mesh,
      scratch_types=(
          [
              pltpu.VMEM((8, BATCH), jnp.int32),
              pltpu.VMEM((8, BATCH), jnp.int32),
              pltpu.VMEM((4, BATCH, 128), jnp.float32),
              pltpu.VMEM_SHARED((NPAD, 128), jnp.float32),
          ]
          + [pltpu.SemaphoreType.DMA] * 16
      ),
  )
  def agg_kernel(h_flat, src1d, dst1d, zeros_h, out_flat,
                 sidx, didx, rows, acc, *sems):
    gsems, ssems, isems = sems[0:4], sems[4:8], sems[8:16]
    cid = lax.axis_index("c")
    sid = lax.axis_index("s")
    row0 = sid * ROWS_PER_TILE
    ebase = sid * EDGES_PER_TILE

    def gstart(s8):
      pltpu.async_copy(h_flat.at[sidx.at[s8]], rows.at[s8 % 4],
                       gsems[s8 % 4])

    def gwait(s4):
      pltpu.make_async_copy(h_flat.at[pl.ds(0, BATCH)], rows.at[s4],
                            gsems[s4]).wait()

    def sstart(s8):
      pltpu.async_copy(rows.at[s8 % 4], acc.at[didx.at[s8]], ssems[s8 % 4],
                       add=True)

    def swait(s4):
      pltpu.make_async_copy(rows.at[s4], acc.at[pl.ds(0, BATCH)],
                            ssems[s4]).wait()

    def istart(s8, b, chunk):
      off = chunk * EPAD + ebase + b * BATCH
      pltpu.async_copy(src1d.at[pl.ds(off, BATCH)], sidx.at[s8], isems[s8])
      pltpu.async_copy(dst1d.at[pl.ds(ebase + b * BATCH, BATCH)],
                       didx.at[s8], isems[s8])

    def iwait(s8):
      for _ in range(2):
        pltpu.make_async_copy(src1d.at[pl.ds(0, BATCH)], sidx.at[s8],
                              isems[s8]).wait()

    for j in range(cpc):
      chunk = cid * cpc + j
      pltpu.sync_copy(zeros_h, acc.at[pl.ds(row0, ROWS_PER_TILE)])
      plsc.subcore_barrier()

      for s in range(8):
        istart(s, s, chunk)
      for s in range(3):
        iwait(s)
        gstart(s)

      def body(t, carry, chunk=chunk):
        for u in range(8):
          b = 8 * t + u
          gwait(u % 4)
          sstart(u)

          if u == 0:
            @pl.when(t >= 1)
            def _(b=b):
              swait(3)
              istart(7, b + 7, chunk)
          else:
            swait((u - 1) % 4)

            @pl.when(t < TRIPS - 1)
            def _(b=b, u=u):
              istart(u - 1, b + 7, chunk)

          if u <= 4:
            iwait((u + 3) % 8)
            gstart((u + 3) % 8)
          else:
            @pl.when(t < TRIPS - 1)
            def _(u=u):
              iwait((u + 3) % 8)
              gstart((u + 3) % 8)
        return carry

      lax.fori_loop(0, TRIPS, body, 0)
      swait((NB - 1) % 4)
      plsc.subcore_barrier()
      pltpu.sync_copy(acc.at[pl.ds(row0, ROWS_PER_TILE)],
                      out_flat.at[pl.ds(chunk * NPAD + row0, ROWS_PER_TILE)])

  return agg_kernel


def _mlp_stats(C):
  fin = C * 128

  @functools.partial(
      pl.pallas_call,
      grid=(NBLK,),
      in_specs=[
          pl.BlockSpec((BR, fin), lambda r: (r, 0)),
          pl.BlockSpec((C, BR, 128), lambda r: (0, r, 0)),
          pl.BlockSpec((fin, DIM), lambda r: (0, 0)),
          pl.BlockSpec((1, DIM), lambda r: (0, 0)),
          pl.BlockSpec((DIM, DIM), lambda r: (0, 0)),
          pl.BlockSpec((1, DIM), lambda r: (0, 0)),
      ],
      out_specs=[
          pl.BlockSpec((BR, DIM), lambda r: (r, 0)),
          pl.BlockSpec((8, DIM), lambda r: (0, 0)),
      ],
      out_shape=[
          jax.ShapeDtypeStruct((N, DIM), jnp.float32),
          jax.ShapeDtypeStruct((8, DIM), jnp.float32),
      ],
  )
  def mlp_kernel(h_ref, agg_ref, w1_ref, b1_ref, w2_ref, b2_ref, y_ref, st_ref):
    r = pl.program_id(0)
    aggf = jnp.concatenate([agg_ref[c] for c in range(C)], axis=1)
    t = jnp.dot(h_ref[...] + aggf, w1_ref[...],
                preferred_element_type=jnp.float32)
    u = jnp.maximum(t + b1_ref[...], 0.0)
    y = jnp.dot(u, w2_ref[...], preferred_element_type=jnp.float32) + b2_ref[...]
    y = jnp.maximum(y, 0.0)
    y_ref[...] = y
    s1 = jnp.sum(y, axis=0, keepdims=True)
    s2 = jnp.sum(y * y, axis=0, keepdims=True)
    upd = jnp.concatenate([s1, s2, jnp.zeros((6, DIM), jnp.float32)], axis=0)

    @pl.when(r == 0)
    def _():
      st_ref[...] = upd

    @pl.when(r != 0)
    def _():
      st_ref[...] += upd

  return mlp_kernel


@functools.partial(
    pl.pallas_call,
    grid=(NBLK,),
    in_specs=[
        pl.BlockSpec((BR, DIM), lambda r: (r, 0)),
        pl.BlockSpec((8, DIM), lambda r: (0, 0)),
        pl.BlockSpec((1, DIM), lambda r: (0, 0)),
        pl.BlockSpec((1, DIM), lambda r: (0, 0)),
        pl.BlockSpec((BR, G), lambda r: (r, 0)),
    ],
    out_specs=[
        pl.BlockSpec((BR, DIM), lambda r: (r, 0)),
        pl.BlockSpec((4, BR, 128), lambda r: (0, r, 0)),
        pl.BlockSpec((G, DIM), lambda r: (0, 0)),
    ],
    out_shape=[
        jax.ShapeDtypeStruct((N, DIM), jnp.float32),
        jax.ShapeDtypeStruct((4, NPAD, 128), jnp.float32),
        jax.ShapeDtypeStruct((G, DIM), jnp.float32),
    ],
)
def _bn_pool(y_ref, st_ref, g_ref, b_ref, pm_ref, hn_ref, slab_ref, pool_ref):
  r = pl.program_id(0)
  s = st_ref[...]
  mean = s[0:1, :] * (1.0 / N)
  var = s[1:2, :] * (1.0 / N) - mean * mean
  scale = g_ref[...] * lax.rsqrt(var + BN_EPS)
  hn = (y_ref[...] - mean) * scale + b_ref[...]
  hn_ref[...] = hn
  for c in range(4):
    slab_ref[c] = hn[:, c * 128:(c + 1) * 128]
  contrib = lax.dot_general(pm_ref[...], hn, (((0,), (0,)), ((), ())),
                            preferred_element_type=jnp.float32)

  @pl.when(r == 0)
  def _():
    pool_ref[...] = contrib

  @pl.when(r != 0)
  def _():
    pool_ref[...] += contrib


def kernel(x, edge_index, batch, w1_0, b1_0, w2_0, b2_0, bn_g_0, bn_b_0,
           w1_1, b1_1, w2_1, b2_1, bn_g_1, bn_b_1,
           w1_2, b1_2, w2_2, b2_2, bn_g_2, bn_b_2):
  f32 = jnp.float32
  src = edge_index[0].astype(jnp.int32)
  dst = edge_index[1].astype(jnp.int32)
  pad = EPAD - E
  src_p = jnp.concatenate([src, jnp.zeros((pad,), jnp.int32)])
  dst_p = jnp.concatenate([dst, jnp.full((pad,), N, jnp.int32)])
  off4 = (jnp.arange(4, dtype=jnp.int32) * NPAD)[:, None]
  src4 = (src_p[None, :] + off4).reshape(-1)
  src2 = src4[:2 * EPAD]
  zeros_h = jnp.zeros((ROWS_PER_TILE, 128), f32)
  pmat = jax.nn.one_hot(batch.astype(jnp.int32), G, dtype=f32)

  x_sl = jnp.pad(x.reshape(N, 2, 128).transpose(1, 0, 2),
                 ((0, 0), (0, NPAD - N), (0, 0))).reshape(2 * NPAD, 128)

  params = [
      (w1_0, b1_0, w2_0, b2_0, bn_g_0, bn_b_0),
      (w1_1, b1_1, w2_1, b2_1, bn_g_1, bn_b_1),
      (w1_2, b1_2, w2_2, b2_2, bn_g_2, bn_b_2),
  ]

  h = x
  h_sl = x_sl
  hs, pools = [], []
  for i, (w1, b1, w2, b2, g, bb) in enumerate(params):
    C = 2 if i == 0 else 4
    srcC = src2 if C == 2 else src4
    agg = _sc_segment_sum(C)(h_sl, srcC, dst_p, zeros_h).reshape(C, NPAD, 128)
    y, st = _mlp_stats(C)(h, agg, w1, b1.reshape(1, DIM), w2, b2.reshape(1, DIM))
    h, slabs, pool = _bn_pool(y, st, g.reshape(1, DIM), bb.reshape(1, DIM), pmat)
    h_sl = slabs.reshape(4 * NPAD, 128)
    hs.append(h)
    pools.append(pool)

  out_pool = jnp.concatenate(pools, axis=1)
  out_nodes = jnp.concatenate(hs, axis=1)
  return (out_pool, out_nodes)


# speedup vs baseline: 2.8647x; 2.8647x over previous
import functools

import jax
import jax.numpy as jnp
from jax import lax
from jax.experimental import pallas as pl
from jax.experimental.pallas import tpu as pltpu
from jax.experimental.pallas import tpu_sc as plsc

N = 10000
NPAD = 10112
ROWS_PER_TILE = NPAD // 16
E = 160000
BATCH = 120
NB = 84
EDGES_PER_TILE = NB * BATCH
EPAD = 16 * EDGES_PER_TILE
TRIPS = NB // 6
DIM = 512
G = 64
BR = 1000
NBLK = N // BR
BN_EPS = 1e-5


def _sc_segment_sum(C):
  mesh = plsc.VectorSubcoreMesh(core_axis_name="c", subcore_axis_name="s")
  cpc = C // 2

  @functools.partial(
      pl.kernel,
      out_type=jax.ShapeDtypeStruct((C * NPAD, 128), jnp.float32),
      mesh=mesh,
      scratch_types=(
          [
              pltpu.VMEM((6, BATCH), jnp.int32),
              pltpu.VMEM((6, BATCH), jnp.int32),
              pltpu.VMEM((3, BATCH, 128), jnp.float32),
              pltpu.VMEM_SHARED((NPAD, 128), jnp.float32),
          ]
          + [pltpu.SemaphoreType.DMA] * 12
      ),
  )
  def agg_kernel(h_flat, src1d, dst1d, zeros_h, out_flat,
                 sidx, didx, rows, acc, *sems):
    gsems, ssems, isems = sems[0:3], sems[3:6], sems[6:12]
    cid = lax.axis_index("c")
    sid = lax.axis_index("s")
    row0 = sid * ROWS_PER_TILE
    ebase = sid * EDGES_PER_TILE

    def gstart(s8):
      pltpu.async_copy(h_flat.at[sidx.at[s8]], rows.at[s8 % 3],
                       gsems[s8 % 3])

    def gwait(s4):
      pltpu.make_async_copy(h_flat.at[pl.ds(0, BATCH)], rows.at[s4],
                            gsems[s4]).wait()

    def sstart(s8):
      pltpu.async_copy(rows.at[s8 % 3], acc.at[didx.at[s8]], ssems[s8 % 3],
                       add=True)

    def swait(s4):
      pltpu.make_async_copy(rows.at[s4], acc.at[pl.ds(0, BATCH)],
                            ssems[s4]).wait()

    def istart(s8, b, chunk):
      off = chunk * EPAD + ebase + b * BATCH
      pltpu.async_copy(src1d.at[pl.ds(off, BATCH)], sidx.at[s8], isems[s8])
      pltpu.async_copy(dst1d.at[pl.ds(ebase + b * BATCH, BATCH)],
                       didx.at[s8], isems[s8])

    def iwait(s8):
      for _ in range(2):
        pltpu.make_async_copy(src1d.at[pl.ds(0, BATCH)], sidx.at[s8],
                              isems[s8]).wait()

    for j in range(cpc):
      chunk = cid * cpc + j
      pltpu.sync_copy(zeros_h, acc.at[pl.ds(row0, ROWS_PER_TILE)])
      plsc.subcore_barrier()

      for s in range(6):
        istart(s, s, chunk)
      for s in range(2):
        iwait(s)
        gstart(s)

      def body(t, carry, chunk=chunk):
        for u in range(6):
          b = 6 * t + u
          gwait(u % 3)
          sstart(u)

          if u == 0:
            @pl.when(t >= 1)
            def _(b=b):
              swait(2)
              istart(5, b + 5, chunk)
          else:
            swait((u - 1) % 3)

            @pl.when(t < TRIPS - 1)
            def _(b=b, u=u):
              istart(u - 1, b + 5, chunk)

          if u <= 3:
            iwait((u + 2) % 6)
            gstart((u + 2) % 6)
          else:
            @pl.when(t < TRIPS - 1)
            def _(u=u):
              iwait((u + 2) % 6)
              gstart((u + 2) % 6)
        return carry

      lax.fori_loop(0, TRIPS, body, 0)
      swait((NB - 1) % 3)
      plsc.subcore_barrier()
      pltpu.sync_copy(acc.at[pl.ds(row0, ROWS_PER_TILE)],
                      out_flat.at[pl.ds(chunk * NPAD + row0, ROWS_PER_TILE)])

  return agg_kernel


def _mlp_stats(C):
  fin = C * 128

  @functools.partial(
      pl.pallas_call,
      grid=(NBLK,),
      in_specs=[
          pl.BlockSpec((BR, fin), lambda r: (r, 0)),
          pl.BlockSpec((C, BR, 128), lambda r: (0, r, 0)),
          pl.BlockSpec((fin, DIM), lambda r: (0, 0)),
          pl.BlockSpec((1, DIM), lambda r: (0, 0)),
          pl.BlockSpec((DIM, DIM), lambda r: (0, 0)),
          pl.BlockSpec((1, DIM), lambda r: (0, 0)),
      ],
      out_specs=[
          pl.BlockSpec((BR, DIM), lambda r: (r, 0)),
          pl.BlockSpec((8, DIM), lambda r: (0, 0)),
      ],
      out_shape=[
          jax.ShapeDtypeStruct((N, DIM), jnp.float32),
          jax.ShapeDtypeStruct((8, DIM), jnp.float32),
      ],
  )
  def mlp_kernel(h_ref, agg_ref, w1_ref, b1_ref, w2_ref, b2_ref, y_ref, st_ref):
    r = pl.program_id(0)
    aggf = jnp.concatenate([agg_ref[c] for c in range(C)], axis=1)
    t = jnp.dot(h_ref[...] + aggf, w1_ref[...],
                preferred_element_type=jnp.float32)
    u = jnp.maximum(t + b1_ref[...], 0.0)
    y = jnp.dot(u, w2_ref[...], preferred_element_type=jnp.float32) + b2_ref[...]
    y = jnp.maximum(y, 0.0)
    y_ref[...] = y
    s1 = jnp.sum(y, axis=0, keepdims=True)
    s2 = jnp.sum(y * y, axis=0, keepdims=True)
    upd = jnp.concatenate([s1, s2, jnp.zeros((6, DIM), jnp.float32)], axis=0)

    @pl.when(r == 0)
    def _():
      st_ref[...] = upd

    @pl.when(r != 0)
    def _():
      st_ref[...] += upd

  return mlp_kernel


@functools.partial(
    pl.pallas_call,
    grid=(NBLK,),
    in_specs=[
        pl.BlockSpec((BR, DIM), lambda r: (r, 0)),
        pl.BlockSpec((8, DIM), lambda r: (0, 0)),
        pl.BlockSpec((1, DIM), lambda r: (0, 0)),
        pl.BlockSpec((1, DIM), lambda r: (0, 0)),
        pl.BlockSpec((BR, G), lambda r: (r, 0)),
    ],
    out_specs=[
        pl.BlockSpec((BR, DIM), lambda r: (r, 0)),
        pl.BlockSpec((4, BR, 128), lambda r: (0, r, 0)),
        pl.BlockSpec((G, DIM), lambda r: (0, 0)),
    ],
    out_shape=[
        jax.ShapeDtypeStruct((N, DIM), jnp.float32),
        jax.ShapeDtypeStruct((4, NPAD, 128), jnp.float32),
        jax.ShapeDtypeStruct((G, DIM), jnp.float32),
    ],
)
def _bn_pool(y_ref, st_ref, g_ref, b_ref, pm_ref, hn_ref, slab_ref, pool_ref):
  r = pl.program_id(0)
  s = st_ref[...]
  mean = s[0:1, :] * (1.0 / N)
  var = s[1:2, :] * (1.0 / N) - mean * mean
  scale = g_ref[...] * lax.rsqrt(var + BN_EPS)
  hn = (y_ref[...] - mean) * scale + b_ref[...]
  hn_ref[...] = hn
  for c in range(4):
    slab_ref[c] = hn[:, c * 128:(c + 1) * 128]
  contrib = lax.dot_general(pm_ref[...], hn, (((0,), (0,)), ((), ())),
                            preferred_element_type=jnp.float32)

  @pl.when(r == 0)
  def _():
    pool_ref[...] = contrib

  @pl.when(r != 0)
  def _():
    pool_ref[...] += contrib


def kernel(x, edge_index, batch, w1_0, b1_0, w2_0, b2_0, bn_g_0, bn_b_0,
           w1_1, b1_1, w2_1, b2_1, bn_g_1, bn_b_1,
           w1_2, b1_2, w2_2, b2_2, bn_g_2, bn_b_2):
  f32 = jnp.float32
  src = edge_index[0].astype(jnp.int32)
  dst = edge_index[1].astype(jnp.int32)
  pad = EPAD - E
  src_p = jnp.concatenate([src, jnp.zeros((pad,), jnp.int32)])
  dst_p = jnp.concatenate([dst, jnp.full((pad,), N, jnp.int32)])
  off4 = (jnp.arange(4, dtype=jnp.int32) * NPAD)[:, None]
  src4 = (src_p[None, :] + off4).reshape(-1)
  src2 = src4[:2 * EPAD]
  zeros_h = jnp.zeros((ROWS_PER_TILE, 128), f32)
  pmat = jax.nn.one_hot(batch.astype(jnp.int32), G, dtype=f32)

  x_sl = jnp.pad(x.reshape(N, 2, 128).transpose(1, 0, 2),
                 ((0, 0), (0, NPAD - N), (0, 0))).reshape(2 * NPAD, 128)

  params = [
      (w1_0, b1_0, w2_0, b2_0, bn_g_0, bn_b_0),
      (w1_1, b1_1, w2_1, b2_1, bn_g_1, bn_b_1),
      (w1_2, b1_2, w2_2, b2_2, bn_g_2, bn_b_2),
  ]

  h = x
  h_sl = x_sl
  hs, pools = [], []
  for i, (w1, b1, w2, b2, g, bb) in enumerate(params):
    C = 2 if i == 0 else 4
    srcC = src2 if C == 2 else src4
    agg = _sc_segment_sum(C)(h_sl, srcC, dst_p, zeros_h).reshape(C, NPAD, 128)
    y, st = _mlp_stats(C)(h, agg, w1, b1.reshape(1, DIM), w2, b2.reshape(1, DIM))
    h, slabs, pool = _bn_pool(y, st, g.reshape(1, DIM), bb.reshape(1, DIM), pmat)
    h_sl = slabs.reshape(4 * NPAD, 128)
    hs.append(h)
    pools.append(pool)

  out_pool = jnp.concatenate(pools, axis=1)
  out_nodes = jnp.concatenate(hs, axis=1)
  return (out_pool, out_nodes)
